# baseline (speedup 1.0000x reference)
import functools

import jax
import jax.numpy as jnp
from jax import lax
from jax.experimental import pallas as pl
from jax.experimental.pallas import tpu as pltpu
from jax.experimental.pallas import tpu_sc as plsc

E = 64
B = 4096
S = 200
BS = B * S
NC = 2
NS = 16
NW = NC * NS
ROWS_PER_W = BS // NW
SEQ_PER_W = ROWS_PER_W // S
L = 16


@jax.jit
def _run(xf, table, pos):
    mesh = plsc.VectorSubcoreMesh(core_axis_name="c", subcore_axis_name="s")

    @functools.partial(
        pl.kernel,
        mesh=mesh,
        compiler_params=pltpu.CompilerParams(use_tc_tiling_on_sc=False),
        out_type=jax.ShapeDtypeStruct((BS, E), jnp.float32),
        scratch_types=[
            pltpu.VMEM((2, S), jnp.int32),
            pltpu.VMEM((2, S, E), jnp.float32),
            pltpu.VMEM((S, E), jnp.float32),
            pltpu.SemaphoreType.DMA,
            pltpu.SemaphoreType.DMA,
        ],
    )
    def body(x_hbm, table_hbm, pos_hbm, out_hbm, idx_v, rows_v, pos_v, sem0, sem1):
        wid = lax.axis_index("s") * NC + lax.axis_index("c")
        base = wid * ROWS_PER_W
        sems = (sem0, sem1)
        pltpu.sync_copy(pos_hbm, pos_v)

        pltpu.sync_copy(x_hbm.at[pl.ds(base, S)], idx_v.at[0])
        cp0 = pltpu.async_copy(table_hbm.at[idx_v.at[0]], rows_v.at[0], sem0)

        def chunk_pair(c, carry):
            for b in range(2):
                cc = c + b
                row0 = base + cc * S

                @pl.when(cc + 1 < SEQ_PER_W)
                def _():
                    nb = 1 - b
                    pltpu.sync_copy(
                        x_hbm.at[pl.ds(row0 + S, S)], idx_v.at[nb])
                    pltpu.async_copy(
                        table_hbm.at[idx_v.at[nb]], rows_v.at[nb], sems[nb])

                pltpu.make_async_copy(
                    table_hbm.at[idx_v.at[b]], rows_v.at[b], sems[b]).wait()

                def add_body(p, carry2):
                    for j in range(E // L):
                        sl = pl.ds(L * j, L)
                        rows_v[b, p, sl] = rows_v[b, p, sl] + pos_v[p, sl]
                    return carry2

                lax.fori_loop(0, S, add_body, 0)
                pltpu.sync_copy(rows_v.at[b], out_hbm.at[pl.ds(row0, S)])
            return carry

        lax.fori_loop(0, SEQ_PER_W // 2, lambda i, c: chunk_pair(i * 2, c), 0)

    return body(xf, table, pos)


def kernel(x, table, pos_encoding):
    xf = x.reshape(-1).astype(jnp.int32)
    pos = pos_encoding[:S]
    out = _run(xf, table, pos)
    return out.reshape(B, S, E)
